# baseline (speedup 1.0000x reference)
import functools

import jax
import jax.numpy as jnp
from jax import lax
from jax.experimental import pallas as pl
from jax.experimental.pallas import tpu as pltpu
from jax.experimental.pallas import tpu_sc as plsc


def _sc_gather(emb, idx, cpw, nw, nc):
    d = emb.shape[1]
    n_rows = idx.shape[0]
    rows_per_w = cpw * 128
    mesh = plsc.VectorSubcoreMesh(core_axis_name="c", subcore_axis_name="s")

    @functools.partial(
        pl.kernel,
        mesh=mesh,
        out_type=jax.ShapeDtypeStruct((n_rows, d), jnp.float32),
        scratch_types=[
            pltpu.VMEM((rows_per_w,), jnp.int32),
            pltpu.VMEM((rows_per_w, d), jnp.float32),
            pltpu.SemaphoreType.DMA,
        ],
        compiler_params=pltpu.CompilerParams(use_tc_tiling_on_sc=False),
    )
    def gather_kernel(emb_hbm, idx_hbm, out_hbm, idx_v, rows_v, sem):
        wid = lax.axis_index("s") * nc + lax.axis_index("c")
        base = wid * rows_per_w
        pltpu.sync_copy(idx_hbm.at[pl.ds(base, rows_per_w)], idx_v)
        copies = []
        for j in range(cpw):
            copies.append(
                pltpu.async_copy(
                    emb_hbm.at[idx_v.at[pl.ds(j * 128, 128)]],
                    rows_v.at[pl.ds(j * 128, 128)],
                    sem,
                )
            )
        for c in copies:
            c.wait()
        pltpu.sync_copy(rows_v, out_hbm.at[pl.ds(base, rows_per_w)])

    return gather_kernel(emb, idx)


def _tc_mlp(flat, w1, b1, w2, b2, bv=2048):
    b, f = flat.shape
    h = w1.shape[0]
    v = w2.shape[0]
    nv = pl.cdiv(v, bv)

    def body(flat_ref, w1_ref, b1_ref, w2_ref, b2_ref, out_ref, h_ref):
        @pl.when(pl.program_id(0) == 0)
        def _():
            acc = lax.dot_general(
                flat_ref[...], w1_ref[...], (((1,), (1,)), ((), ())),
                preferred_element_type=jnp.float32)
            h_ref[...] = jnp.maximum(acc + b1_ref[...], 0.0)

        out_ref[...] = lax.dot_general(
            h_ref[...], w2_ref[...], (((1,), (1,)), ((), ())),
            preferred_element_type=jnp.float32) + b2_ref[...]

    return pl.pallas_call(
        body,
        grid=(nv,),
        in_specs=[
            pl.BlockSpec((b, f), lambda j: (0, 0)),
            pl.BlockSpec((h, f), lambda j: (0, 0)),
            pl.BlockSpec((1, h), lambda j: (0, 0)),
            pl.BlockSpec((bv, h), lambda j: (j, 0)),
            pl.BlockSpec((1, bv), lambda j: (0, j)),
        ],
        out_specs=pl.BlockSpec((b, bv), lambda j: (0, j)),
        out_shape=jax.ShapeDtypeStruct((b, v), jnp.float32),
        scratch_shapes=[pltpu.VMEM((b, h), jnp.float32)],
    )(flat, w1, b1.reshape(1, h), w2, b2.reshape(1, v))


def kernel(x, emb, W1, b1, W2, b2):
    batch, ctx = x.shape
    d = emb.shape[1]
    n = batch * ctx

    info = plsc.get_sparse_core_info()
    nc, ns = info.num_cores, info.num_subcores
    nw = nc * ns
    assert n % (128 * nw) == 0
    cpw = n // (128 * nw)

    idx = x.reshape(n).astype(jnp.int32)
    rows = _sc_gather(emb, idx, cpw, nw, nc)
    flat = rows.reshape(batch, ctx * d)
    return _tc_mlp(flat, W1, b1, W2, b2)


# speedup vs baseline: 1.0073x; 1.0073x over previous
import functools

import jax
import jax.numpy as jnp
from jax import lax
from jax.experimental import pallas as pl
from jax.experimental.pallas import tpu as pltpu
from jax.experimental.pallas import tpu_sc as plsc


def _sc_gather(emb, idx, cpw, nw, nc):
    d = emb.shape[1]
    n_rows = idx.shape[0]
    rows_per_w = cpw * 128
    mesh = plsc.VectorSubcoreMesh(core_axis_name="c", subcore_axis_name="s")

    @functools.partial(
        pl.kernel,
        mesh=mesh,
        out_type=jax.ShapeDtypeStruct((n_rows, d), jnp.float32),
        scratch_types=[
            pltpu.VMEM((rows_per_w,), jnp.int32),
            pltpu.VMEM((rows_per_w, d), jnp.float32),
            pltpu.SemaphoreType.DMA,
        ],
        compiler_params=pltpu.CompilerParams(use_tc_tiling_on_sc=False),
    )
    def gather_kernel(emb_hbm, idx_hbm, out_hbm, idx_v, rows_v, sem):
        wid = lax.axis_index("s") * nc + lax.axis_index("c")
        base = wid * rows_per_w
        pltpu.sync_copy(idx_hbm.at[pl.ds(base, rows_per_w)], idx_v)
        copies = []
        for j in range(cpw):
            copies.append(
                pltpu.async_copy(
                    emb_hbm.at[idx_v.at[pl.ds(j * 128, 128)]],
                    rows_v.at[pl.ds(j * 128, 128)],
                    sem,
                )
            )
        for c in copies:
            c.wait()
        pltpu.sync_copy(rows_v, out_hbm.at[pl.ds(base, rows_per_w)])

    return gather_kernel(emb, idx)


def _tc_mlp(flat, w1, b1, w2, b2, bv=4096):
    b, f = flat.shape
    h = w1.shape[0]
    v = w2.shape[0]
    nv = pl.cdiv(v, bv)

    def body(flat_ref, w1_ref, b1_ref, w2_ref, b2_ref, out_ref, h_ref):
        @pl.when(pl.program_id(0) == 0)
        def _():
            acc = lax.dot_general(
                flat_ref[...], w1_ref[...], (((1,), (1,)), ((), ())),
                preferred_element_type=jnp.float32)
            h_ref[...] = jnp.maximum(acc + b1_ref[...], 0.0)

        out_ref[...] = lax.dot_general(
            h_ref[...], w2_ref[...], (((1,), (1,)), ((), ())),
            preferred_element_type=jnp.float32) + b2_ref[...]

    return pl.pallas_call(
        body,
        grid=(nv,),
        in_specs=[
            pl.BlockSpec((b, f), lambda j: (0, 0)),
            pl.BlockSpec((h, f), lambda j: (0, 0)),
            pl.BlockSpec((1, h), lambda j: (0, 0)),
            pl.BlockSpec((bv, h), lambda j: (j, 0)),
            pl.BlockSpec((1, bv), lambda j: (0, j)),
        ],
        out_specs=pl.BlockSpec((b, bv), lambda j: (0, j)),
        out_shape=jax.ShapeDtypeStruct((b, v), jnp.float32),
        scratch_shapes=[pltpu.VMEM((b, h), jnp.float32)],
    )(flat, w1, b1.reshape(1, h), w2, b2.reshape(1, v))


def kernel(x, emb, W1, b1, W2, b2):
    batch, ctx = x.shape
    d = emb.shape[1]
    n = batch * ctx

    info = plsc.get_sparse_core_info()
    nc, ns = info.num_cores, info.num_subcores
    nw = nc * ns
    assert n % (128 * nw) == 0
    cpw = n // (128 * nw)

    idx = x.reshape(n).astype(jnp.int32)
    rows = _sc_gather(emb, idx, cpw, nw, nc)
    flat = rows.reshape(batch, ctx * d)
    return _tc_mlp(flat, W1, b1, W2, b2)


# speedup vs baseline: 2.3375x; 2.3206x over previous
import functools

import jax
import jax.numpy as jnp
from jax import lax
from jax.experimental import pallas as pl
from jax.experimental.pallas import tpu as pltpu
from jax.experimental.pallas import tpu_sc as plsc


def _sc_gather(emb, idx, cpw, nw, nc):
    d = emb.shape[1]
    n_rows = idx.shape[0]
    rows_per_w = cpw * 128
    mesh = plsc.VectorSubcoreMesh(core_axis_name="c", subcore_axis_name="s")

    @functools.partial(
        pl.kernel,
        mesh=mesh,
        out_type=jax.ShapeDtypeStruct((n_rows, d), jnp.float32),
        scratch_types=[
            pltpu.VMEM((rows_per_w,), jnp.int32),
            pltpu.VMEM((rows_per_w, d), jnp.float32),
            pltpu.SemaphoreType.DMA,
        ],
        compiler_params=pltpu.CompilerParams(use_tc_tiling_on_sc=False),
    )
    def gather_kernel(emb_hbm, idx_hbm, out_hbm, idx_v, rows_v, sem):
        wid = lax.axis_index("s") * nc + lax.axis_index("c")
        base = wid * rows_per_w
        pltpu.sync_copy(idx_hbm.at[pl.ds(base, rows_per_w)], idx_v)
        copies = []
        for j in range(cpw):
            copies.append(
                pltpu.async_copy(
                    emb_hbm.at[idx_v.at[pl.ds(j * 128, 128)]],
                    rows_v.at[pl.ds(j * 128, 128)],
                    sem,
                )
            )
        for c in copies:
            c.wait()
        pltpu.sync_copy(rows_v, out_hbm.at[pl.ds(base, rows_per_w)])

    return gather_kernel(emb, idx)


def _tc_mlp_t(flat, w1, b1, w2t, b2c, bv=2048):
    b, f = flat.shape
    h = w1.shape[0]
    v = w2t.shape[1]
    ns = pl.cdiv(v, bv)

    def body(flat_ref, w1_ref, b1_ref, w2t_ref, b2_ref, out_ref, h_ref):
        @pl.when(pl.program_id(0) == 0)
        def _():
            acc = lax.dot_general(
                flat_ref[...], w1_ref[...], (((1,), (1,)), ((), ())),
                preferred_element_type=jnp.float32)
            h_ref[...] = jnp.maximum(acc + b1_ref[...], 0.0)

        acc = lax.dot_general(
            w2t_ref[...], h_ref[...], (((0,), (1,)), ((), ())),
            preferred_element_type=jnp.float32)
        out_ref[...] = acc + b2_ref[...]

    return pl.pallas_call(
        body,
        grid=(ns,),
        in_specs=[
            pl.BlockSpec((b, f), lambda j: (0, 0)),
            pl.BlockSpec((h, f), lambda j: (0, 0)),
            pl.BlockSpec((1, h), lambda j: (0, 0)),
            pl.BlockSpec((h, bv), lambda j: (0, j)),
            pl.BlockSpec((bv, 1), lambda j: (j, 0)),
        ],
        out_specs=pl.BlockSpec((bv, b), lambda j: (j, 0)),
        out_shape=jax.ShapeDtypeStruct((v, b), jnp.float32),
        scratch_shapes=[pltpu.VMEM((b, h), jnp.float32)],
    )(flat, w1, b1.reshape(1, h), w2t, b2c)


def kernel(x, emb, W1, b1, W2, b2):
    batch, ctx = x.shape
    d = emb.shape[1]
    n = batch * ctx

    info = plsc.get_sparse_core_info()
    nc, ns = info.num_cores, info.num_subcores
    nw = nc * ns
    assert n % (128 * nw) == 0
    cpw = n // (128 * nw)

    idx = x.reshape(n).astype(jnp.int32)
    rows = _sc_gather(emb, idx, cpw, nw, nc)
    flat = rows.reshape(batch, ctx * d)
    out_t = _tc_mlp_t(flat, W1, b1, W2.T, b2.reshape(-1, 1))
    return out_t.T


# speedup vs baseline: 2.9012x; 1.2412x over previous
import functools

import jax
import jax.numpy as jnp
from jax import lax
from jax.experimental import pallas as pl
from jax.experimental.pallas import tpu as pltpu
from jax.experimental.pallas import tpu_sc as plsc


def _sc_gather(emb, idx, cpw, nw, nc):
    d = emb.shape[1]
    n_rows = idx.shape[0]
    rows_per_w = cpw * 128
    mesh = plsc.VectorSubcoreMesh(core_axis_name="c", subcore_axis_name="s")

    @functools.partial(
        pl.kernel,
        mesh=mesh,
        out_type=jax.ShapeDtypeStruct((n_rows, d), jnp.float32),
        scratch_types=[
            pltpu.VMEM((rows_per_w,), jnp.int32),
            pltpu.VMEM((rows_per_w, d), jnp.float32),
            pltpu.SemaphoreType.DMA,
        ],
        compiler_params=pltpu.CompilerParams(use_tc_tiling_on_sc=False),
    )
    def gather_kernel(emb_hbm, idx_hbm, out_hbm, idx_v, rows_v, sem):
        wid = lax.axis_index("s") * nc + lax.axis_index("c")
        base = wid * rows_per_w
        pltpu.sync_copy(idx_hbm.at[pl.ds(base, rows_per_w)], idx_v)
        copies = []
        for j in range(cpw):
            copies.append(
                pltpu.async_copy(
                    emb_hbm.at[idx_v.at[pl.ds(j * 128, 128)]],
                    rows_v.at[pl.ds(j * 128, 128)],
                    sem,
                )
            )
        for c in copies:
            c.wait()
        pltpu.sync_copy(rows_v, out_hbm.at[pl.ds(base, rows_per_w)])

    return gather_kernel(emb, idx)


def _tc_mlp_t(flat, w1, b1, w2t, b2, bv=2048):
    b, f = flat.shape
    h = w1.shape[0]
    v = w2t.shape[1]
    ns = pl.cdiv(v, bv)

    def body(flat_ref, w1_ref, b1_ref, w2t_ref, b2_ref, out_ref, h_ref):
        @pl.when(pl.program_id(0) == 0)
        def _():
            acc = lax.dot_general(
                flat_ref[...], w1_ref[...], (((1,), (1,)), ((), ())),
                preferred_element_type=jnp.float32)
            h_ref[...] = jnp.maximum(acc + b1_ref[...], 0.0)

        acc = lax.dot_general(
            w2t_ref[...], h_ref[...], (((0,), (1,)), ((), ())),
            preferred_element_type=jnp.float32)
        out_ref[...] = acc + b2_ref[...][:, None]

    return pl.pallas_call(
        body,
        grid=(ns,),
        in_specs=[
            pl.BlockSpec((b, f), lambda j: (0, 0)),
            pl.BlockSpec((h, f), lambda j: (0, 0)),
            pl.BlockSpec((1, h), lambda j: (0, 0)),
            pl.BlockSpec((h, bv), lambda j: (0, j)),
            pl.BlockSpec((bv,), lambda j: (j,)),
        ],
        out_specs=pl.BlockSpec((bv, b), lambda j: (j, 0)),
        out_shape=jax.ShapeDtypeStruct((v, b), jnp.float32),
        scratch_shapes=[pltpu.VMEM((b, h), jnp.float32)],
    )(flat, w1, b1.reshape(1, h), w2t, b2)


def kernel(x, emb, W1, b1, W2, b2):
    batch, ctx = x.shape
    d = emb.shape[1]
    n = batch * ctx

    info = plsc.get_sparse_core_info()
    nc, ns = info.num_cores, info.num_subcores
    nw = nc * ns
    assert n % (128 * nw) == 0
    cpw = n // (128 * nw)

    idx = x.reshape(n).astype(jnp.int32)
    rows = _sc_gather(emb, idx, cpw, nw, nc)
    flat = rows.reshape(batch, ctx * d)
    out_t = _tc_mlp_t(flat, W1, b1, W2.T, b2)
    return out_t.T


# speedup vs baseline: 2.9532x; 1.0179x over previous
import functools

import jax
import jax.numpy as jnp
from jax import lax
from jax.experimental import pallas as pl
from jax.experimental.pallas import tpu as pltpu
from jax.experimental.pallas import tpu_sc as plsc


def _sc_gather(emb, idx, cpw, nw, nc):
    d = emb.shape[1]
    n_rows = idx.shape[0]
    rows_per_w = cpw * 128
    mesh = plsc.VectorSubcoreMesh(core_axis_name="c", subcore_axis_name="s")

    @functools.partial(
        pl.kernel,
        mesh=mesh,
        out_type=jax.ShapeDtypeStruct((n_rows, d), jnp.float32),
        scratch_types=[
            pltpu.VMEM((rows_per_w,), jnp.int32),
            pltpu.VMEM((rows_per_w, d), jnp.float32),
            pltpu.SemaphoreType.DMA,
        ],
        compiler_params=pltpu.CompilerParams(use_tc_tiling_on_sc=False),
    )
    def gather_kernel(emb_hbm, idx_hbm, out_hbm, idx_v, rows_v, sem):
        wid = lax.axis_index("s") * nc + lax.axis_index("c")
        base = wid * rows_per_w
        pltpu.sync_copy(idx_hbm.at[pl.ds(base, rows_per_w)], idx_v)
        copies = []
        for j in range(cpw):
            copies.append(
                pltpu.async_copy(
                    emb_hbm.at[idx_v.at[pl.ds(j * 128, 128)]],
                    rows_v.at[pl.ds(j * 128, 128)],
                    sem,
                )
            )
        for c in copies:
            c.wait()
        pltpu.sync_copy(rows_v, out_hbm.at[pl.ds(base, rows_per_w)])

    return gather_kernel(emb, idx)


def _tc_mlp_t(flat, w1, b1, w2t, b2, bv=4096):
    b, f = flat.shape
    h = w1.shape[0]
    v = w2t.shape[1]
    ns = pl.cdiv(v, bv)

    def body(flat_ref, w1_ref, b1_ref, w2t_ref, b2_ref, out_ref, h_ref):
        @pl.when(pl.program_id(0) == 0)
        def _():
            acc = lax.dot_general(
                flat_ref[...], w1_ref[...], (((1,), (1,)), ((), ())),
                preferred_element_type=jnp.float32)
            h_ref[...] = jnp.maximum(acc + b1_ref[...], 0.0)

        acc = lax.dot_general(
            w2t_ref[...], h_ref[...], (((0,), (1,)), ((), ())),
            preferred_element_type=jnp.float32)
        out_ref[...] = acc + b2_ref[...][:, None]

    return pl.pallas_call(
        body,
        grid=(ns,),
        in_specs=[
            pl.BlockSpec((b, f), lambda j: (0, 0)),
            pl.BlockSpec((h, f), lambda j: (0, 0)),
            pl.BlockSpec((1, h), lambda j: (0, 0)),
            pl.BlockSpec((h, bv), lambda j: (0, j)),
            pl.BlockSpec((bv,), lambda j: (j,)),
        ],
        out_specs=pl.BlockSpec((bv, b), lambda j: (j, 0)),
        out_shape=jax.ShapeDtypeStruct((v, b), jnp.float32),
        scratch_shapes=[pltpu.VMEM((b, h), jnp.float32)],
    )(flat, w1, b1.reshape(1, h), w2t, b2)


def kernel(x, emb, W1, b1, W2, b2):
    batch, ctx = x.shape
    d = emb.shape[1]
    n = batch * ctx

    info = plsc.get_sparse_core_info()
    nc, ns = info.num_cores, info.num_subcores
    nw = nc * ns
    assert n % (128 * nw) == 0
    cpw = n // (128 * nw)

    idx = x.reshape(n).astype(jnp.int32)
    rows = _sc_gather(emb, idx, cpw, nw, nc)
    flat = rows.reshape(batch, ctx * d)
    out_t = _tc_mlp_t(flat, W1, b1, W2.T, b2)
    return out_t.T


# speedup vs baseline: 2.9543x; 1.0004x over previous
import functools

import jax
import jax.numpy as jnp
from jax import lax
from jax.experimental import pallas as pl
from jax.experimental.pallas import tpu as pltpu
from jax.experimental.pallas import tpu_sc as plsc


def _sc_gather(emb, idx, cpw, nw, nc):
    d = emb.shape[1]
    n_rows = idx.shape[0]
    rows_per_w = cpw * 128
    mesh = plsc.VectorSubcoreMesh(core_axis_name="c", subcore_axis_name="s")

    @functools.partial(
        pl.kernel,
        mesh=mesh,
        out_type=jax.ShapeDtypeStruct((n_rows, d), jnp.float32),
        scratch_types=[
            pltpu.VMEM((rows_per_w,), jnp.int32),
            pltpu.VMEM((rows_per_w, d), jnp.float32),
            pltpu.SemaphoreType.DMA,
        ],
        compiler_params=pltpu.CompilerParams(use_tc_tiling_on_sc=False),
    )
    def gather_kernel(emb_hbm, idx_hbm, out_hbm, idx_v, rows_v, sem):
        wid = lax.axis_index("s") * nc + lax.axis_index("c")
        base = wid * rows_per_w
        pltpu.sync_copy(idx_hbm.at[pl.ds(base, rows_per_w)], idx_v)
        copies = []
        for j in range(cpw):
            copies.append(
                pltpu.async_copy(
                    emb_hbm.at[idx_v.at[pl.ds(j * 128, 128)]],
                    rows_v.at[pl.ds(j * 128, 128)],
                    sem,
                )
            )
        for c in copies:
            c.wait()
        pltpu.sync_copy(rows_v, out_hbm.at[pl.ds(base, rows_per_w)])

    return gather_kernel(emb, idx)


def _tc_mlp_t(flat, w1, b1, w2t, b2, bv=4096):
    b, f = flat.shape
    h = w1.shape[0]
    v = w2t.shape[1]
    ns = pl.cdiv(v, bv)

    def body(flat_ref, w1_ref, b1_ref, w2t_ref, b2_ref, out_ref, ht_ref):
        @pl.when(pl.program_id(0) == 0)
        def _():
            acc = lax.dot_general(
                w1_ref[...], flat_ref[...], (((1,), (1,)), ((), ())),
                preferred_element_type=jnp.float32)
            ht_ref[...] = jnp.maximum(acc + b1_ref[...][:, None], 0.0)

        acc = lax.dot_general(
            w2t_ref[...], ht_ref[...], (((0,), (0,)), ((), ())),
            preferred_element_type=jnp.float32)
        out_ref[...] = acc + b2_ref[...][:, None]

    return pl.pallas_call(
        body,
        grid=(ns,),
        in_specs=[
            pl.BlockSpec((b, f), lambda j: (0, 0)),
            pl.BlockSpec((h, f), lambda j: (0, 0)),
            pl.BlockSpec((h,), lambda j: (0,)),
            pl.BlockSpec((h, bv), lambda j: (0, j)),
            pl.BlockSpec((bv,), lambda j: (j,)),
        ],
        out_specs=pl.BlockSpec((bv, b), lambda j: (j, 0)),
        out_shape=jax.ShapeDtypeStruct((v, b), jnp.float32),
        scratch_shapes=[pltpu.VMEM((h, b), jnp.float32)],
    )(flat, w1, b1, w2t, b2)


def kernel(x, emb, W1, b1, W2, b2):
    batch, ctx = x.shape
    d = emb.shape[1]
    n = batch * ctx

    info = plsc.get_sparse_core_info()
    nc, ns = info.num_cores, info.num_subcores
    nw = nc * ns
    assert n % (128 * nw) == 0
    cpw = n // (128 * nw)

    idx = x.reshape(n).astype(jnp.int32)
    rows = _sc_gather(emb, idx, cpw, nw, nc)
    flat = rows.reshape(batch, ctx * d)
    out_t = _tc_mlp_t(flat, W1, b1, W2.T, b2)
    return out_t.T
